# baseline (speedup 1.0000x reference)
import functools

import jax
import jax.numpy as jnp
from jax import lax
from jax.experimental import pallas as pl
from jax.experimental.pallas import tpu as pltpu
from jax.experimental.pallas import tpu_sc as plsc

HIDDEN = 256
NUM_EMB_ROWS = 95
NUM_ROWS = 100000
CHUNK = 160
NCHUNKS = NUM_ROWS // CHUNK
NC, NS = 2, 16
NW = NC * NS
ITERS_W = -(-NCHUNKS // NW)
NLAST = NCHUNKS - (ITERS_W - 1) * NW

_mesh = plsc.VectorSubcoreMesh(core_axis_name="c", subcore_axis_name="s")


@functools.partial(
    pl.kernel,
    out_type=jax.ShapeDtypeStruct((NUM_ROWS, HIDDEN), jnp.int32),
    mesh=_mesh,
    scratch_types=[
        pltpu.VMEM((CHUNK,), jnp.int32),
        pltpu.VMEM((CHUNK,), jnp.int32),
        pltpu.VMEM((CHUNK, 128), jnp.int32),
        pltpu.VMEM((CHUNK, 128), jnp.int32),
        pltpu.VMEM((CHUNK, HIDDEN), jnp.int32),
        pltpu.VMEM((CHUNK, HIDDEN), jnp.int32),
        pltpu.SemaphoreType.DMA,
        pltpu.SemaphoreType.DMA,
        pltpu.SemaphoreType.DMA,
        pltpu.SemaphoreType.DMA,
        pltpu.SemaphoreType.DMA,
        pltpu.SemaphoreType.DMA,
    ],
)
def _emb_lookup(x_hbm, tab_hbm, out_hbm, idx0, idx1, bf0, bf1, f0, f1,
                i0, i1, g0, g1, s0, s1):
    wid = lax.axis_index("s") * NC + lax.axis_index("c")
    idx = (idx0, idx1)
    bfs = (bf0, bf1)
    fs = (f0, f1)
    isem = (i0, i1)
    gsem = (g0, g1)
    ssem = (s0, s1)

    def load_idx(j):
        b = j & 1
        base = (wid + j * NW) * CHUNK
        return pltpu.async_copy(x_hbm.at[pl.ds(base, CHUNK)], idx[b], isem[b])

    def start_gather(j):
        b = j & 1
        return pltpu.async_copy(tab_hbm.at[idx[b]], bfs[b], gsem[b])

    def start_store(j):
        b = j & 1
        base = (wid + j * NW) * CHUNK
        return pltpu.async_copy(fs[b], out_hbm.at[pl.ds(base, CHUNK)], ssem[b])

    def convert(j):
        b = j & 1

        def row_body(r, carry):
            for c in range(8):
                u = bfs[b][r, pl.ds(c * 16, 16)]
                fs[b][r, pl.ds(c * 32, 16)] = u << 16
                fs[b][r, pl.ds(c * 32 + 16, 16)] = u & jnp.int32(-65536)
            return carry

        lax.fori_loop(0, CHUNK, row_body, 0)

    last = ITERS_W - 1
    idx_d = [None] * ITERS_W
    gd = [None] * ITERS_W
    sd = [None] * ITERS_W

    idx_d[0] = load_idx(0)
    idx_d[1] = load_idx(1)
    idx_d[0].wait()
    gd[0] = start_gather(0)

    for j in range(ITERS_W - 1):
        gd[j].wait()
        if j + 2 < last:
            idx_d[j + 2] = load_idx(j + 2)
        elif j + 2 == last:
            @pl.when(wid < NLAST)
            def _():
                load_idx(last)
        if j + 1 < last:
            idx_d[j + 1].wait()
            gd[j + 1] = start_gather(j + 1)
        else:
            @pl.when(wid < NLAST)
            def _():
                b = last & 1
                pltpu.make_async_copy(
                    x_hbm.at[pl.ds(0, CHUNK)], idx[b], isem[b]).wait()
                start_gather(last)
        if j >= 2:
            sd[j - 2].wait()
        convert(j)
        sd[j] = start_store(j)

    @pl.when(wid < NLAST)
    def _():
        b = last & 1
        pltpu.make_async_copy(tab_hbm.at[idx[b]], bfs[b], gsem[b]).wait()
        sd[last - 2].wait()
        convert(last)
        start_store(last).wait()

    @pl.when(wid >= NLAST)
    def _():
        sd[last - 2].wait()

    sd[last - 1].wait()


def kernel(x, emb_weight):
    copy_id = (jnp.arange(NUM_ROWS, dtype=jnp.int32) // CHUNK) % NW
    x_adj = x.astype(jnp.int32) + NUM_EMB_ROWS * copy_id
    wb = emb_weight.astype(jnp.bfloat16)
    wb = wb.reshape(NUM_EMB_ROWS, 8, 2, 16).transpose(0, 1, 3, 2)
    wi = jax.lax.bitcast_convert_type(
        wb.reshape(NUM_EMB_ROWS, 128, 2), jnp.int32)
    tab_rep = jnp.tile(wi, (NW, 1))
    out_bits = _emb_lookup(x_adj, tab_rep)
    return jax.lax.bitcast_convert_type(out_bits, jnp.float32)


# speedup vs baseline: 1.3592x; 1.3592x over previous
import functools

import jax
import jax.numpy as jnp
from jax import lax
from jax.experimental import pallas as pl
from jax.experimental.pallas import tpu as pltpu
from jax.experimental.pallas import tpu_sc as plsc

HIDDEN = 256
NUM_EMB_ROWS = 95
NUM_ROWS = 100000
CHUNK = 160
NCHUNKS = NUM_ROWS // CHUNK
NC, NS = 2, 16
NW = NC * NS
ITERS_W = -(-NCHUNKS // NW)
NLAST = NCHUNKS - (ITERS_W - 1) * NW

_mesh = plsc.VectorSubcoreMesh(core_axis_name="c", subcore_axis_name="s")


@functools.partial(
    pl.kernel,
    out_type=jax.ShapeDtypeStruct((NUM_ROWS, HIDDEN), jnp.int32),
    mesh=_mesh,
    scratch_types=[
        pltpu.VMEM((CHUNK,), jnp.int32),
        pltpu.VMEM((CHUNK,), jnp.int32),
        pltpu.VMEM((CHUNK, 128), jnp.int32),
        pltpu.VMEM((CHUNK, 128), jnp.int32),
        pltpu.VMEM((CHUNK, HIDDEN), jnp.int32),
        pltpu.VMEM((CHUNK, HIDDEN), jnp.int32),
        pltpu.SemaphoreType.DMA,
        pltpu.SemaphoreType.DMA,
        pltpu.SemaphoreType.DMA,
        pltpu.SemaphoreType.DMA,
        pltpu.SemaphoreType.DMA,
        pltpu.SemaphoreType.DMA,
    ],
)
def _emb_lookup(x_hbm, tab_hbm, out_hbm, idx0, idx1, bf0, bf1, f0, f1,
                i0, i1, g0, g1, s0, s1):
    wid = lax.axis_index("s") * NC + lax.axis_index("c")
    idx = (idx0, idx1)
    bfs = (bf0, bf1)
    fs = (f0, f1)
    isem = (i0, i1)
    gsem = (g0, g1)
    ssem = (s0, s1)

    def load_idx(j):
        b = j & 1
        base = (wid + j * NW) * CHUNK
        return pltpu.async_copy(x_hbm.at[pl.ds(base, CHUNK)], idx[b], isem[b])

    def start_gather(j):
        b = j & 1
        return pltpu.async_copy(tab_hbm.at[idx[b]], bfs[b], gsem[b])

    def start_store(j):
        b = j & 1
        base = (wid + j * NW) * CHUNK
        return pltpu.async_copy(fs[b], out_hbm.at[pl.ds(base, CHUNK)], ssem[b])

    def convert(j):
        b = j & 1

        @plsc.parallel_loop(0, CHUNK, unroll=4)
        def row_body(r):
            for c in range(8):
                u = bfs[b][r, pl.ds(c * 16, 16)]
                fs[b][r, pl.ds(c * 32, 16)] = u << 16
                fs[b][r, pl.ds(c * 32 + 16, 16)] = u & jnp.int32(-65536)

    last = ITERS_W - 1
    idx_d = [None] * ITERS_W
    gd = [None] * ITERS_W
    sd = [None] * ITERS_W

    idx_d[0] = load_idx(0)
    idx_d[1] = load_idx(1)
    idx_d[0].wait()
    gd[0] = start_gather(0)

    for j in range(ITERS_W - 1):
        gd[j].wait()
        if j + 2 < last:
            idx_d[j + 2] = load_idx(j + 2)
        elif j + 2 == last:
            @pl.when(wid < NLAST)
            def _():
                load_idx(last)
        if j + 1 < last:
            idx_d[j + 1].wait()
            gd[j + 1] = start_gather(j + 1)
        else:
            @pl.when(wid < NLAST)
            def _():
                b = last & 1
                pltpu.make_async_copy(
                    x_hbm.at[pl.ds(0, CHUNK)], idx[b], isem[b]).wait()
                start_gather(last)
        if j >= 2:
            sd[j - 2].wait()
        convert(j)
        sd[j] = start_store(j)

    @pl.when(wid < NLAST)
    def _():
        b = last & 1
        pltpu.make_async_copy(tab_hbm.at[idx[b]], bfs[b], gsem[b]).wait()
        sd[last - 2].wait()
        convert(last)
        start_store(last).wait()

    @pl.when(wid >= NLAST)
    def _():
        sd[last - 2].wait()

    sd[last - 1].wait()


def kernel(x, emb_weight):
    copy_id = (jnp.arange(NUM_ROWS, dtype=jnp.int32) // CHUNK) % NW
    x_adj = x.astype(jnp.int32) + NUM_EMB_ROWS * copy_id
    wb = emb_weight.astype(jnp.bfloat16)
    wb = wb.reshape(NUM_EMB_ROWS, 8, 2, 16).transpose(0, 1, 3, 2)
    wi = jax.lax.bitcast_convert_type(
        wb.reshape(NUM_EMB_ROWS, 128, 2), jnp.int32)
    tab_rep = jnp.tile(wi, (NW, 1))
    out_bits = _emb_lookup(x_adj, tab_rep)
    return jax.lax.bitcast_convert_type(out_bits, jnp.float32)


# speedup vs baseline: 2.0344x; 1.4968x over previous
import functools

import jax
import jax.numpy as jnp
from jax import lax
from jax.experimental import pallas as pl
from jax.experimental.pallas import tpu as pltpu
from jax.experimental.pallas import tpu_sc as plsc

HIDDEN = 256
NUM_EMB_ROWS = 95
NUM_ROWS = 100000
CHUNK = 200
NCHUNKS = NUM_ROWS // CHUNK
NC, NS = 2, 16
NW = NC * NS
ITERS_W = -(-NCHUNKS // NW)

_mesh = plsc.VectorSubcoreMesh(core_axis_name="c", subcore_axis_name="s")


@functools.partial(
    pl.kernel,
    out_type=jax.ShapeDtypeStruct((NUM_ROWS, HIDDEN), jnp.float32),
    mesh=_mesh,
    scratch_types=[
        pltpu.VMEM((CHUNK,), jnp.int32),
        pltpu.VMEM((CHUNK,), jnp.int32),
        pltpu.VMEM((CHUNK, HIDDEN), jnp.float32),
        pltpu.VMEM((CHUNK, HIDDEN), jnp.float32),
        pltpu.SemaphoreType.DMA,
        pltpu.SemaphoreType.DMA,
        pltpu.SemaphoreType.DMA,
        pltpu.SemaphoreType.DMA,
    ],
)
def _emb_lookup(x_hbm, tab_hbm, out_hbm, idx0, idx1, rows0, rows1,
                g0, g1, s0, s1):
    wid = lax.axis_index("s") * NC + lax.axis_index("c")
    idx = (idx0, idx1)
    rows = (rows0, rows1)
    gsem = (g0, g1)
    ssem = (s0, s1)

    def start_gather(j):
        b = j & 1
        base = (wid + j * NW) * CHUNK
        pltpu.sync_copy(x_hbm.at[pl.ds(base, CHUNK)], idx[b])
        return pltpu.async_copy(tab_hbm.at[idx[b]], rows[b], gsem[b])

    def start_store(j):
        b = j & 1
        base = (wid + j * NW) * CHUNK
        return pltpu.async_copy(rows[b], out_hbm.at[pl.ds(base, CHUNK)], ssem[b])

    gd = [None] * ITERS_W
    sd = [None] * ITERS_W
    for j in range(ITERS_W - 1):
        if j >= 2:
            sd[j - 2].wait()
        gd[j] = start_gather(j)
        if j >= 1:
            gd[j - 1].wait()
            sd[j - 1] = start_store(j - 1)

    last = ITERS_W - 1
    gd[last - 1].wait()
    sd[last - 1] = start_store(last - 1)
    sd[last - 2].wait()

    @pl.when(wid + last * NW < NCHUNKS)
    def _():
        start_gather(last).wait()
        start_store(last).wait()

    sd[last - 1].wait()


def kernel(x, emb_weight):
    copy_id = (jnp.arange(NUM_ROWS, dtype=jnp.int32) // CHUNK) % NW
    x_adj = x.astype(jnp.int32) + NUM_EMB_ROWS * copy_id
    tab_rep = jnp.tile(emb_weight, (NW, 1))
    return _emb_lookup(x_adj, tab_rep)
